# baseline (speedup 1.0000x reference)
import functools
import math

import jax
import jax.numpy as jnp
from jax.experimental import pallas as pl

NH = 16


def _qkv_kernel(x_ref, w_ref, b_ref, o_ref, *, heads_per_step, hs):
    r = jnp.dot(x_ref[...], w_ref[...], preferred_element_type=jnp.float32)
    r = r + b_ref[...]
    for hh in range(heads_per_step):
        o_ref[0, hh] = r[:, hh * hs:(hh + 1) * hs]


def _attn_kernel(q_ref, k_ref, v_ref, o_ref, *, bq, nq, scale):
    hs = q_ref.shape[3]
    rows = jax.lax.broadcasted_iota(jnp.int32, (bq, bq), 0)
    cols = jax.lax.broadcasted_iota(jnp.int32, (bq, bq), 1)
    log2e_scale = scale * 1.4426950408889634
    for qi in range(nq):
        q = q_ref[0, 0, qi * bq:(qi + 1) * bq, :] * log2e_scale
        s_blocks = []
        m_acc = None
        for j in range(qi + 1):
            kj = k_ref[0, 0, j * bq:(j + 1) * bq, :]
            s = jax.lax.dot_general(q, kj, (((1,), (1,)), ((), ())),
                                    preferred_element_type=jnp.float32)
            if j == qi:
                s = jnp.where(cols <= rows, s, -1e30)
            s_blocks.append(s)
            m_acc = s if m_acc is None else jnp.maximum(m_acc, s)
        m = jnp.max(m_acc, axis=1, keepdims=True)
        l_acc = jnp.zeros((bq, bq), dtype=jnp.float32)
        acc = jnp.zeros((bq, hs), dtype=jnp.float32)
        for j in range(qi + 1):
            p = jnp.exp2(s_blocks[j] - m)
            l_acc = l_acc + p
            vj = v_ref[0, 0, j * bq:(j + 1) * bq, :]
            acc = acc + jnp.dot(p, vj, preferred_element_type=jnp.float32)
        l = jnp.sum(l_acc, axis=1, keepdims=True)
        o_ref[qi * bq:(qi + 1) * bq, :] = acc * (1.0 / l)


def kernel(x, W_attn, b_attn, W_proj, b_proj):
    B, T, C = x.shape
    HS = C // NH
    G = 3 * NH

    x2 = x.reshape(B * T, C)

    heads_per_step = 4
    bn1 = heads_per_step * HS
    ng1 = G // heads_per_step
    qkv = pl.pallas_call(
        functools.partial(_qkv_kernel, heads_per_step=heads_per_step, hs=HS),
        grid=(B, ng1),
        in_specs=[
            pl.BlockSpec((T, C), lambda b, j: (b, 0)),
            pl.BlockSpec((C, bn1), lambda b, j: (0, j)),
            pl.BlockSpec((1, bn1), lambda b, j: (0, j)),
        ],
        out_specs=pl.BlockSpec((1, heads_per_step, T, HS),
                               lambda b, j: (b, j, 0, 0)),
        out_shape=jax.ShapeDtypeStruct((B, G, T, HS), jnp.float32),
    )(x2, W_attn, b_attn.reshape(1, 3 * C))

    BQ = 256
    nq = T // BQ
    scale = 1.0 / math.sqrt(HS)
    y = pl.pallas_call(
        functools.partial(_attn_kernel, bq=BQ, nq=nq, scale=scale),
        grid=(B, NH),
        in_specs=[
            pl.BlockSpec((1, 1, T, HS), lambda b, h: (b, h, 0, 0)),
            pl.BlockSpec((1, 1, T, HS), lambda b, h: (b, NH + h, 0, 0)),
            pl.BlockSpec((1, 1, T, HS), lambda b, h: (b, 2 * NH + h, 0, 0)),
        ],
        out_specs=pl.BlockSpec((T, HS), lambda b, h: (b, h)),
        out_shape=jax.ShapeDtypeStruct((B * T, C), jnp.float32),
    )(qkv, qkv, qkv)

    bn3 = 512
    nn3 = C // bn3
    out = pl.pallas_call(
        functools.partial(_qkv_kernel, heads_per_step=1, hs=bn3),
        grid=(B, nn3),
        in_specs=[
            pl.BlockSpec((T, C), lambda b, j: (b, 0)),
            pl.BlockSpec((C, bn3), lambda b, j: (0, j)),
            pl.BlockSpec((1, bn3), lambda b, j: (0, j)),
        ],
        out_specs=pl.BlockSpec((1, 1, T, bn3), lambda b, j: (b, 0, 0, j)),
        out_shape=jax.ShapeDtypeStruct((B, 1, T, C), jnp.float32),
    )(y, W_proj, b_proj.reshape(1, C))

    return out.reshape(B, T, C)


# speedup vs baseline: 1.0251x; 1.0251x over previous
import functools
import math

import jax
import jax.numpy as jnp
from jax.experimental import pallas as pl

NH = 16


def _qkv_kernel(x_ref, w_ref, b_ref, o_ref, *, heads_per_step, hs):
    r = jnp.dot(x_ref[...], w_ref[...], preferred_element_type=jnp.float32)
    r = r + b_ref[...]
    for hh in range(heads_per_step):
        o_ref[0, hh] = r[:, hh * hs:(hh + 1) * hs]


def _attn_kernel(q_ref, k_ref, v_ref, o_ref, *, bq, nq, scale):
    hs = q_ref.shape[3]
    rows = jax.lax.broadcasted_iota(jnp.int32, (bq, bq), 0)
    cols = jax.lax.broadcasted_iota(jnp.int32, (bq, bq), 1)
    log2e_scale = scale * 1.4426950408889634
    for qi in range(nq):
        q = q_ref[0, 0, qi * bq:(qi + 1) * bq, :] * log2e_scale
        s_blocks = []
        m_acc = None
        for j in range(qi + 1):
            kj = k_ref[0, 0, j * bq:(j + 1) * bq, :]
            s = jax.lax.dot_general(q, kj, (((1,), (1,)), ((), ())),
                                    preferred_element_type=jnp.float32)
            if j == qi:
                s = jnp.where(cols <= rows, s, -1e30)
            s_blocks.append(s)
            m_acc = s if m_acc is None else jnp.maximum(m_acc, s)
        m = jnp.max(m_acc, axis=1, keepdims=True)
        l_acc = jnp.zeros((bq, bq), dtype=jnp.float32)
        acc = jnp.zeros((bq, hs), dtype=jnp.float32)
        for j in range(qi + 1):
            p = jnp.exp2(s_blocks[j] - m)
            l_acc = l_acc + p
            vj = v_ref[0, 0, j * bq:(j + 1) * bq, :]
            acc = acc + jnp.dot(p, vj, preferred_element_type=jnp.float32)
        l = jnp.sum(l_acc, axis=1, keepdims=True)
        o_ref[qi * bq:(qi + 1) * bq, :] = acc * (1.0 / l)


def kernel(x, W_attn, b_attn, W_proj, b_proj):
    B, T, C = x.shape
    HS = C // NH
    G = 3 * NH

    x2 = x.reshape(B * T, C)

    heads_per_step = 4
    bn1 = heads_per_step * HS
    ng1 = G // heads_per_step
    qkv = pl.pallas_call(
        functools.partial(_qkv_kernel, heads_per_step=heads_per_step, hs=HS),
        grid=(B, ng1),
        in_specs=[
            pl.BlockSpec((T, C), lambda b, j: (b, 0)),
            pl.BlockSpec((C, bn1), lambda b, j: (0, j)),
            pl.BlockSpec((1, bn1), lambda b, j: (0, j)),
        ],
        out_specs=pl.BlockSpec((1, heads_per_step, T, HS),
                               lambda b, j: (b, j, 0, 0)),
        out_shape=jax.ShapeDtypeStruct((B, G, T, HS), jnp.float32),
    )(x2, W_attn, b_attn.reshape(1, 3 * C))

    BQ = 512
    nq = T // BQ
    scale = 1.0 / math.sqrt(HS)
    y = pl.pallas_call(
        functools.partial(_attn_kernel, bq=BQ, nq=nq, scale=scale),
        grid=(B, NH),
        in_specs=[
            pl.BlockSpec((1, 1, T, HS), lambda b, h: (b, h, 0, 0)),
            pl.BlockSpec((1, 1, T, HS), lambda b, h: (b, NH + h, 0, 0)),
            pl.BlockSpec((1, 1, T, HS), lambda b, h: (b, 2 * NH + h, 0, 0)),
        ],
        out_specs=pl.BlockSpec((T, HS), lambda b, h: (b, h)),
        out_shape=jax.ShapeDtypeStruct((B * T, C), jnp.float32),
    )(qkv, qkv, qkv)

    bn3 = 512
    nn3 = C // bn3
    out = pl.pallas_call(
        functools.partial(_qkv_kernel, heads_per_step=1, hs=bn3),
        grid=(B, nn3),
        in_specs=[
            pl.BlockSpec((T, C), lambda b, j: (b, 0)),
            pl.BlockSpec((C, bn3), lambda b, j: (0, j)),
            pl.BlockSpec((1, bn3), lambda b, j: (0, j)),
        ],
        out_specs=pl.BlockSpec((1, 1, T, bn3), lambda b, j: (b, 0, 0, j)),
        out_shape=jax.ShapeDtypeStruct((B, 1, T, C), jnp.float32),
    )(y, W_proj, b_proj.reshape(1, C))

    return out.reshape(B, T, C)
